# baseline (speedup 1.0000x reference)
import functools

import jax
import jax.numpy as jnp
from jax import lax
from jax.experimental import pallas as pl
from jax.experimental.pallas import tpu as pltpu
from jax.experimental.pallas import tpu_sc as plsc

_B = 4096
_L = 2048
_BB = 512
_BSC = 1024
_BTC = _B - _BSC


def _onehot_block(seq_ref, out_ref):
    s = seq_ref[...]
    for i in range(4):
        out_ref[:, i, :] = (s == i).astype(jnp.float32)


def _tc_onehot(seq, rows):
    return pl.pallas_call(
        _onehot_block,
        grid=(rows // _BB,),
        in_specs=[pl.BlockSpec((_BB, _L), lambda i: (i, 0))],
        out_specs=pl.BlockSpec((_BB, 4, _L), lambda i: (i, 0, 0)),
        out_shape=jax.ShapeDtypeStruct((rows, 4, _L), jnp.float32),
    )(seq)


_NC = 2
_NS = 16
_NW = _NC * _NS
_RC = 4
_NSLICE = _L // 16


def _make_sc_onehot(rows):
    rpw = rows // _NW
    nchunk = rpw // _RC

    @functools.partial(
        pl.kernel,
        mesh=plsc.VectorSubcoreMesh(core_axis_name="c", subcore_axis_name="s"),
        out_type=jax.ShapeDtypeStruct((rows, 4, _L), jnp.float32),
        scratch_types=[
            pltpu.VMEM((_RC, _L), jnp.int32),
            pltpu.VMEM((_RC, 4, _L), jnp.float32),
        ],
    )
    def _sc_onehot(seq_hbm, out_hbm, seq_v, out_v):
        wid = lax.axis_index("s") * _NC + lax.axis_index("c")
        base = wid * rpw

        def chunk_body(c, carry):
            row0 = base + c * _RC
            pltpu.sync_copy(seq_hbm.at[pl.ds(row0, _RC)], seq_v)

            def slice_body(j, carry2):
                off = j * 16
                for r in range(_RC):
                    s = seq_v[r, pl.ds(off, 16)]
                    for i in range(4):
                        out_v[r, i, pl.ds(off, 16)] = jnp.where(
                            s == i, jnp.float32(1.0), jnp.float32(0.0)
                        )
                return carry2

            lax.fori_loop(0, _NSLICE, slice_body, 0, unroll=False)
            pltpu.sync_copy(out_v, out_hbm.at[pl.ds(row0, _RC)])
            return carry

        lax.fori_loop(0, nchunk, chunk_body, 0, unroll=False)

    return _sc_onehot


_sc_onehot_top = _make_sc_onehot(_BSC)


def kernel(sequences):
    seq = sequences.astype(jnp.int32)
    sc_part = _sc_onehot_top(seq[:_BSC])
    tc_part = _tc_onehot(seq[_BSC:], _BTC)
    return jnp.concatenate([sc_part, tc_part], axis=0)


# speedup vs baseline: 1.8193x; 1.8193x over previous
import functools

import jax
import jax.numpy as jnp
from jax import lax
from jax.experimental import pallas as pl
from jax.experimental.pallas import tpu as pltpu
from jax.experimental.pallas import tpu_sc as plsc

_B = 4096
_L = 2048

_NC = 2
_NS = 16
_NW = _NC * _NS
_RC = 4
_NSLICE = _L // 16
_RPW = _B // _NW
_NCHUNK = _RPW // _RC


@functools.partial(
    pl.kernel,
    mesh=plsc.VectorSubcoreMesh(core_axis_name="c", subcore_axis_name="s"),
    out_type=jax.ShapeDtypeStruct((_B, 4, _L), jnp.float32),
    scratch_types=[
        pltpu.VMEM((2, _RC, _L), jnp.int32),
        pltpu.VMEM((2, _RC, 4, _L), jnp.float32),
        pltpu.SemaphoreType.DMA,
        pltpu.SemaphoreType.DMA,
        pltpu.SemaphoreType.DMA,
        pltpu.SemaphoreType.DMA,
    ],
)
def _sc_onehot(seq_hbm, out_hbm, seq_v, out_v, sin0, sin1, sout0, sout1):
    wid = lax.axis_index("s") * _NC + lax.axis_index("c")
    base = wid * _RPW
    sins = (sin0, sin1)
    souts = (sout0, sout1)

    def start_in(c, b):
        row0 = base + c * _RC
        pltpu.make_async_copy(
            seq_hbm.at[pl.ds(row0, _RC)], seq_v.at[b], sins[b]
        ).start()

    def wait_in(c, b):
        row0 = base + c * _RC
        pltpu.make_async_copy(
            seq_hbm.at[pl.ds(row0, _RC)], seq_v.at[b], sins[b]
        ).wait()

    def start_out(c, b):
        row0 = base + c * _RC
        pltpu.make_async_copy(
            out_v.at[b], out_hbm.at[pl.ds(row0, _RC)], souts[b]
        ).start()

    def wait_out(c, b):
        row0 = base + c * _RC
        pltpu.make_async_copy(
            out_v.at[b], out_hbm.at[pl.ds(row0, _RC)], souts[b]
        ).wait()

    start_in(0, 0)
    start_in(1, 1)

    def pair_body(cp, carry):
        c0 = cp * 2
        for b in range(2):
            c = c0 + b
            wait_in(c, b)

            @pl.when(c >= 2)
            def _():
                wait_out(c - 2, b)

            def slice_body(j, carry2):
                off = j * 16
                for r in range(_RC):
                    s = seq_v[b, r, pl.ds(off, 16)]
                    for i in range(4):
                        out_v[b, r, i, pl.ds(off, 16)] = jnp.where(
                            s == i, jnp.float32(1.0), jnp.float32(0.0)
                        )
                return carry2

            lax.fori_loop(0, _NSLICE, slice_body, 0, unroll=2)
            start_out(c, b)

            @pl.when(c + 2 < _NCHUNK)
            def _():
                start_in(c + 2, b)

        return carry

    lax.fori_loop(0, _NCHUNK // 2, pair_body, 0, unroll=False)
    wait_out(_NCHUNK - 2, 0)
    wait_out(_NCHUNK - 1, 1)


def kernel(sequences):
    seq = sequences.astype(jnp.int32)
    return _sc_onehot(seq)
